# Initial kernel scaffold; baseline (speedup 1.0000x reference)
import jax
import jax.numpy as jnp
from jax.experimental import pallas as pl


def kernel(x, pos_emb):
    raise NotImplementedError("write your pallas kernel here")


# speedup vs baseline: 1.7149x; 1.7149x over previous
import jax
import jax.numpy as jnp
from jax.experimental import pallas as pl

BLOCK_ROWS = 256


def _add_body(x_ref, pos_ref, out_ref):
    out_ref[...] = x_ref[...] + pos_ref[...][None, :, :]


def kernel(x, pos_emb):
    batch, n_rows, dim = x.shape
    grid = (n_rows // BLOCK_ROWS,)
    return pl.pallas_call(
        _add_body,
        grid=grid,
        in_specs=[
            pl.BlockSpec((batch, BLOCK_ROWS, dim), lambda i: (0, i, 0)),
            pl.BlockSpec((BLOCK_ROWS, dim), lambda i: (i, 0)),
        ],
        out_specs=pl.BlockSpec((batch, BLOCK_ROWS, dim), lambda i: (0, i, 0)),
        out_shape=jax.ShapeDtypeStruct(x.shape, x.dtype),
    )(x, pos_emb)
